# Initial kernel scaffold; baseline (speedup 1.0000x reference)
import jax
import jax.numpy as jnp
from jax.experimental import pallas as pl


def kernel(x, edge_index, W1, wih1, whh1, bih1, bhh1, W2, wih2, whh2, bih2, bhh2, W3, wih3, whh3, bih3, bhh3):
    raise NotImplementedError("write your pallas kernel here")


# speedup vs baseline: 3.2142x; 3.2142x over previous
import functools

import numpy as np

import jax
import jax.numpy as jnp
from jax import lax
from jax.experimental import pallas as pl
from jax.experimental.pallas import tpu as pltpu
from jax.experimental.pallas import tpu_sc as plsc

N = 10000
C = 128
E = 320000
L = 10

NSC = 2
NTILE = 16
NW = NSC * NTILE
K = 128
NCH = 79
EPT_PAD = NCH * K
NGRP = K // 16
NR = N + 32
ROWS_PER_TILE = 624
ZTAIL = NR - NTILE * ROWS_PER_TILE
OTAIL = N - NTILE * ROWS_PER_TILE

_CHUNK_SIZES = np.array(([10080] * 11 + [9840] * 4 + [9760]) * 2, np.int64)
_BOUNDS = np.concatenate([[0], np.cumsum(_CHUNK_SIZES)])


def _sc_agg_body(m_h, pk_h, zero_h, frow_h, out_h,
                 pkbuf, rbuf, stash_buf, frow_v, agg_sh, gsem):
    c = lax.axis_index("c")
    s = lax.axis_index("s")
    wid = c * NTILE + s

    r0 = s * ROWS_PER_TILE
    pltpu.sync_copy(zero_h.at[pl.ds(r0, ROWS_PER_TILE)],
                    agg_sh.at[pl.ds(r0, ROWS_PER_TILE)])

    @pl.when(s == NTILE - 1)
    def _():
        pltpu.sync_copy(zero_h.at[pl.ds(NTILE * ROWS_PER_TILE, ZTAIL)],
                        agg_sh.at[pl.ds(NTILE * ROWS_PER_TILE, ZTAIL)])

    plsc.subcore_barrier()

    zvec = jnp.zeros((16,), jnp.float32)

    def window(w, acc):
        pltpu.sync_copy(pk_h.at[wid, w], pkbuf)
        pltpu.async_copy(m_h.at[pkbuf.at[0]], rbuf, gsem).wait()

        def group(g, acc):
            samev = pkbuf[2, pl.ds(g * 16, 16)]
            for e in range(16):
                row = g * 16 + e
                idx = jnp.full((16,), e, jnp.int32)
                same_e = lax.gather(
                    samev, idx[:, None],
                    lax.GatherDimensionNumbers(
                        offset_dims=(), collapsed_slice_dims=(0,),
                        start_index_map=(0,)),
                    (1,), mode=lax.GatherScatterMode.PROMISE_IN_BOUNDS)
                sf = same_e.astype(jnp.float32)
                new_acc = []
                for j in range(8):
                    rj = rbuf[row, pl.ds(16 * j, 16)]
                    aj = acc[j] * sf + rj
                    rbuf[row, pl.ds(16 * j, 16)] = aj
                    new_acc.append(aj)
                acc = tuple(new_acc)
            return acc

        acc = lax.fori_loop(0, NGRP, group, acc)
        pltpu.sync_copy(rbuf, agg_sh.at[pkbuf.at[1]])
        return acc

    lax.fori_loop(0, NCH, window, (zvec,) * 8)
    plsc.subcore_barrier()

    @pl.when(s == 0)
    def _():
        pltpu.sync_copy(agg_sh.at[pl.ds(N + 16, 16)], stash_buf)
        pltpu.sync_copy(frow_h.at[c], frow_v)
        pltpu.sync_copy(stash_buf, agg_sh.at[frow_v], add=True)

    plsc.subcore_barrier()

    pltpu.sync_copy(agg_sh.at[pl.ds(r0, ROWS_PER_TILE)],
                    out_h.at[c, pl.ds(r0, ROWS_PER_TILE)])

    @pl.when(s == NTILE - 1)
    def _():
        pltpu.sync_copy(agg_sh.at[pl.ds(NTILE * ROWS_PER_TILE, OTAIL)],
                        out_h.at[c, pl.ds(NTILE * ROWS_PER_TILE, OTAIL)])


_sc_agg = pl.kernel(
    _sc_agg_body,
    out_type=jax.ShapeDtypeStruct((NSC, N, C), jnp.float32),
    mesh=plsc.VectorSubcoreMesh(core_axis_name="c", subcore_axis_name="s",
                                num_cores=NSC, num_subcores=NTILE),
    scratch_types=[
        pltpu.VMEM((3, K), jnp.int32),
        pltpu.VMEM((K, C), jnp.float32),
        pltpu.VMEM((16, C), jnp.float32),
        pltpu.VMEM((16,), jnp.int32),
        pltpu.VMEM_SHARED((NR, C), jnp.float32),
        pltpu.SemaphoreType.DMA,
    ],
)


def _mm_body(x_ref, w_ref, o_ref):
    o_ref[...] = jnp.dot(x_ref[...], w_ref[...],
                         preferred_element_type=jnp.float32)


_mm = pl.pallas_call(
    _mm_body,
    grid=(5,),
    in_specs=[pl.BlockSpec((2000, C), lambda i: (i, 0)),
              pl.BlockSpec((C, C), lambda i: (0, 0))],
    out_specs=pl.BlockSpec((2000, C), lambda i: (i, 0)),
    out_shape=jax.ShapeDtypeStruct((N, C), jnp.float32),
)


def _gi_body(y0_ref, y1_ref, w_ref, b_ref, o_ref):
    agg = y0_ref[...] + y1_ref[...]
    o_ref[...] = lax.dot_general(agg, w_ref[...], (((1,), (1,)), ((), ())),
                                 preferred_element_type=jnp.float32) + b_ref[...]


_gi = pl.pallas_call(
    _gi_body,
    grid=(5,),
    in_specs=[pl.BlockSpec((2000, C), lambda i: (i, 0)),
              pl.BlockSpec((2000, C), lambda i: (i, 0)),
              pl.BlockSpec((3 * C, C), lambda i: (0, 0)),
              pl.BlockSpec((1, 3 * C), lambda i: (0, 0))],
    out_specs=pl.BlockSpec((2000, 3 * C), lambda i: (i, 0)),
    out_shape=jax.ShapeDtypeStruct((N, 3 * C), jnp.float32),
)


def _gh_body(x_ref, w_ref, b_ref, o_ref):
    o_ref[...] = lax.dot_general(x_ref[...], w_ref[...],
                                 (((1,), (1,)), ((), ())),
                                 preferred_element_type=jnp.float32) + b_ref[...]


_gh = pl.pallas_call(
    _gh_body,
    grid=(5,),
    in_specs=[pl.BlockSpec((2000, C), lambda i: (i, 0)),
              pl.BlockSpec((3 * C, C), lambda i: (0, 0)),
              pl.BlockSpec((1, 3 * C), lambda i: (0, 0))],
    out_specs=pl.BlockSpec((2000, 3 * C), lambda i: (i, 0)),
    out_shape=jax.ShapeDtypeStruct((N, 3 * C), jnp.float32),
)


def _gates_body(relu, gi_ref, gh_ref, x_ref, o_ref):
    gi = gi_ref[...]
    gh = gh_ref[...]
    h = x_ref[...]
    r = jax.nn.sigmoid(gi[:, 0:C] + gh[:, 0:C])
    z = jax.nn.sigmoid(gi[:, C:2 * C] + gh[:, C:2 * C])
    n = jnp.tanh(gi[:, 2 * C:] + r * gh[:, 2 * C:])
    out = (1.0 - z) * n + z * h
    if relu:
        out = jnp.maximum(out, 0.0)
    o_ref[...] = out


def _make_gates(relu):
    return pl.pallas_call(
        functools.partial(_gates_body, relu),
        grid=(5,),
        in_specs=[pl.BlockSpec((2000, 3 * C), lambda i: (i, 0)),
                  pl.BlockSpec((2000, 3 * C), lambda i: (i, 0)),
                  pl.BlockSpec((2000, C), lambda i: (i, 0))],
        out_specs=pl.BlockSpec((2000, C), lambda i: (i, 0)),
        out_shape=jax.ShapeDtypeStruct((N, C), jnp.float32),
    )


_gates = _make_gates(False)
_gates_relu = _make_gates(True)


def _build_plans(edge_index):
    src = edge_index[0]
    dst = edge_index[1]
    perm = jnp.argsort(dst, stable=True)
    ssrc = src[perm]
    sdst = dst[perm]

    bounds = jnp.asarray(_BOUNDS, jnp.int32)
    e = jnp.arange(E, dtype=jnp.int32)
    t_of_e = jnp.searchsorted(bounds, e, side="right").astype(jnp.int32) - 1
    pos = e - bounds[t_of_e]
    gpos = t_of_e * EPT_PAD + pos

    chunk_start = jnp.zeros((E,), jnp.bool_).at[bounds[:-1]].set(True)
    chunk_end = jnp.zeros((E,), jnp.bool_).at[bounds[1:] - 1].set(True)
    same = jnp.concatenate([jnp.zeros((1,), jnp.bool_),
                            sdst[1:] == sdst[:-1]]) & ~chunk_start
    islast = jnp.concatenate([sdst[1:] != sdst[:-1],
                              jnp.ones((1,), jnp.bool_)]) | chunk_end

    t = jnp.arange(NW, dtype=jnp.int32)
    bt = bounds[:-1]
    cont = (bt % (E // NSC) != 0) & (sdst[bt] == sdst[jnp.maximum(bt - 1, 0)])
    first_row = sdst[bt]
    sac = N + (t % NTILE)
    stash = N + NTILE + (t % NTILE)

    redirect = cont[t_of_e] & (sdst == first_row[t_of_e])
    target = jnp.where(redirect, stash[t_of_e], sdst)
    rowid_e = jnp.where(islast, target, sac[t_of_e])

    flat = NW * EPT_PAD
    fpos = jnp.arange(flat, dtype=jnp.int32)
    pad_t = fpos // EPT_PAD
    gidx = ((fpos * 2003) % N).at[gpos].set(ssrc)
    rowid = (N + (pad_t % NTILE)).at[gpos].set(rowid_e)
    same_p = jnp.zeros((flat,), jnp.int32).at[gpos].set(same.astype(jnp.int32))

    pk = jnp.stack([gidx.reshape(NW, NCH, K),
                    rowid.reshape(NW, NCH, K),
                    same_p.reshape(NW, NCH, K)], axis=2)
    frow = jnp.where(cont, first_row, sac).reshape(NSC, NTILE)
    return pk, frow


def kernel(x, edge_index,
           W1, wih1, whh1, bih1, bhh1,
           W2, wih2, whh2, bih2, bhh2,
           W3, wih3, whh3, bih3, bhh3):
    pk, frow = _build_plans(edge_index)
    zeros = jnp.zeros((NR, C), jnp.float32)

    layers = (
        (W1, wih1, whh1, bih1, bhh1, True),
        (W2, wih2, whh2, bih2, bhh2, True),
        (W3, wih3, whh3, bih3, bhh3, False),
    )
    for (W, wih, whh, bih, bhh, relu_at_end) in layers:
        bih2d = bih.reshape(1, 3 * C)
        bhh2d = bhh.reshape(1, 3 * C)
        for i in range(L):
            m = _mm(x, W[i])
            y = _sc_agg(m, pk, zeros, frow)
            gi = _gi(y[0], y[1], wih, bih2d)
            gh = _gh(x, whh, bhh2d)
            gates = _gates_relu if (relu_at_end and i == L - 1) else _gates
            x = gates(gi, gh, x)
    return x


# speedup vs baseline: 3.5829x; 1.1147x over previous
import functools

import numpy as np

import jax
import jax.numpy as jnp
from jax import lax
from jax.experimental import pallas as pl
from jax.experimental.pallas import tpu as pltpu
from jax.experimental.pallas import tpu_sc as plsc

N = 10000
C = 128
E = 320000
L = 10

NSC = 2
NTILE = 16
NW = NSC * NTILE
K = 128
NCH = 80
EPT_PAD = NCH * K
NGRP = K // 16
NR = N + 32
ROWS_PER_TILE = 624
ZTAIL = NR - NTILE * ROWS_PER_TILE
OTAIL = N - NTILE * ROWS_PER_TILE

_CHUNK_SIZES = np.array(([10080] * 11 + [9840] * 4 + [9760]) * 2, np.int64)
_BOUNDS = np.concatenate([[0], np.cumsum(_CHUNK_SIZES)])


def _sc_agg_body(m_h, pk_h, zero_h, frow_h, out_h,
                 pkbuf, rbuf, stash_buf, frow_v, agg_sh,
                 gsem0, gsem1, ssem0, ssem1):
    c = lax.axis_index("c")
    s = lax.axis_index("s")
    wid = c * NTILE + s

    r0 = s * ROWS_PER_TILE
    pltpu.sync_copy(zero_h.at[pl.ds(r0, ROWS_PER_TILE)],
                    agg_sh.at[pl.ds(r0, ROWS_PER_TILE)])

    @pl.when(s == NTILE - 1)
    def _():
        pltpu.sync_copy(zero_h.at[pl.ds(NTILE * ROWS_PER_TILE, ZTAIL)],
                        agg_sh.at[pl.ds(NTILE * ROWS_PER_TILE, ZTAIL)])

    plsc.subcore_barrier()

    zvec = jnp.zeros((16,), jnp.float32)
    pks = [pkbuf.at[i] for i in range(4)]
    rbufs = [rbuf.at[0], rbuf.at[1]]
    gsems = [gsem0, gsem1]
    ssems = [ssem0, ssem1]

    def compute(pk, rb, acc):
        def group(g, acc):
            samev = pk[2, pl.ds(g * 16, 16)]
            for e in range(16):
                row = g * 16 + e
                idx = jnp.full((16,), e, jnp.int32)
                same_e = lax.gather(
                    samev, idx[:, None],
                    lax.GatherDimensionNumbers(
                        offset_dims=(), collapsed_slice_dims=(0,),
                        start_index_map=(0,)),
                    (1,), mode=lax.GatherScatterMode.PROMISE_IN_BOUNDS)
                sf = same_e.astype(jnp.float32)
                new_acc = []
                for j in range(8):
                    rj = rb[row, pl.ds(16 * j, 16)]
                    aj = acc[j] * sf + rj
                    rb[row, pl.ds(16 * j, 16)] = aj
                    new_acc.append(aj)
                acc = tuple(new_acc)
            return acc

        return lax.fori_loop(0, NGRP, group, acc)

    pltpu.sync_copy(pk_h.at[wid, 0], pks[0])
    pltpu.async_copy(m_h.at[pks[0].at[0]], rbufs[0], gsems[0])

    def quad(w2, acc):
        for b4 in range(4):
            w = w2 * 4 + b4
            b = b4 % 2
            pk = pks[b4]
            rb = rbufs[b]
            pltpu.make_async_copy(m_h.at[pk.at[0]], rb, gsems[b]).wait()
            acc = compute(pk, rb, acc)
            pltpu.async_copy(rb, agg_sh.at[pk.at[1]], ssems[b])

            @pl.when(w + 1 < NCH)
            def _():
                @pl.when(w >= 1)
                def _():
                    pltpu.make_async_copy(
                        rbufs[1 - b], agg_sh.at[pks[(b4 + 3) % 4].at[1]],
                        ssems[1 - b]).wait()

                pltpu.sync_copy(pk_h.at[wid, w + 1], pks[(b4 + 1) % 4])
                pltpu.async_copy(m_h.at[pks[(b4 + 1) % 4].at[0]],
                                 rbufs[1 - b], gsems[1 - b])

        return acc

    lax.fori_loop(0, NCH // 4, quad, (zvec,) * 8)
    pltpu.make_async_copy(rbufs[0], agg_sh.at[pks[2].at[1]], ssems[0]).wait()
    pltpu.make_async_copy(rbufs[1], agg_sh.at[pks[3].at[1]], ssems[1]).wait()
    plsc.subcore_barrier()

    @pl.when(s == 0)
    def _():
        pltpu.sync_copy(agg_sh.at[pl.ds(N + 16, 16)], stash_buf)
        pltpu.sync_copy(frow_h.at[c], frow_v)
        pltpu.sync_copy(stash_buf, agg_sh.at[frow_v], add=True)

    plsc.subcore_barrier()

    pltpu.sync_copy(agg_sh.at[pl.ds(r0, ROWS_PER_TILE)],
                    out_h.at[c, pl.ds(r0, ROWS_PER_TILE)])

    @pl.when(s == NTILE - 1)
    def _():
        pltpu.sync_copy(agg_sh.at[pl.ds(NTILE * ROWS_PER_TILE, OTAIL)],
                        out_h.at[c, pl.ds(NTILE * ROWS_PER_TILE, OTAIL)])


_sc_agg = pl.kernel(
    _sc_agg_body,
    out_type=jax.ShapeDtypeStruct((NSC, N, C), jnp.float32),
    mesh=plsc.VectorSubcoreMesh(core_axis_name="c", subcore_axis_name="s",
                                num_cores=NSC, num_subcores=NTILE),
    scratch_types=[
        pltpu.VMEM((4, 3, K), jnp.int32),
        pltpu.VMEM((2, K, C), jnp.float32),
        pltpu.VMEM((16, C), jnp.float32),
        pltpu.VMEM((16,), jnp.int32),
        pltpu.VMEM_SHARED((NR, C), jnp.float32),
        pltpu.SemaphoreType.DMA,
        pltpu.SemaphoreType.DMA,
        pltpu.SemaphoreType.DMA,
        pltpu.SemaphoreType.DMA,
    ],
)


def _mm_body(x_ref, w_ref, o_ref):
    o_ref[...] = jnp.dot(x_ref[...], w_ref[...],
                         preferred_element_type=jnp.float32)


_mm = pl.pallas_call(
    _mm_body,
    grid=(5,),
    in_specs=[pl.BlockSpec((2000, C), lambda i: (i, 0)),
              pl.BlockSpec((C, C), lambda i: (0, 0))],
    out_specs=pl.BlockSpec((2000, C), lambda i: (i, 0)),
    out_shape=jax.ShapeDtypeStruct((N, C), jnp.float32),
)


def _gi_body(y0_ref, y1_ref, w_ref, b_ref, o_ref):
    agg = y0_ref[...] + y1_ref[...]
    o_ref[...] = lax.dot_general(agg, w_ref[...], (((1,), (1,)), ((), ())),
                                 preferred_element_type=jnp.float32) + b_ref[...]


_gi = pl.pallas_call(
    _gi_body,
    grid=(5,),
    in_specs=[pl.BlockSpec((2000, C), lambda i: (i, 0)),
              pl.BlockSpec((2000, C), lambda i: (i, 0)),
              pl.BlockSpec((3 * C, C), lambda i: (0, 0)),
              pl.BlockSpec((1, 3 * C), lambda i: (0, 0))],
    out_specs=pl.BlockSpec((2000, 3 * C), lambda i: (i, 0)),
    out_shape=jax.ShapeDtypeStruct((N, 3 * C), jnp.float32),
)


def _gh_body(x_ref, w_ref, b_ref, o_ref):
    o_ref[...] = lax.dot_general(x_ref[...], w_ref[...],
                                 (((1,), (1,)), ((), ())),
                                 preferred_element_type=jnp.float32) + b_ref[...]


_gh = pl.pallas_call(
    _gh_body,
    grid=(5,),
    in_specs=[pl.BlockSpec((2000, C), lambda i: (i, 0)),
              pl.BlockSpec((3 * C, C), lambda i: (0, 0)),
              pl.BlockSpec((1, 3 * C), lambda i: (0, 0))],
    out_specs=pl.BlockSpec((2000, 3 * C), lambda i: (i, 0)),
    out_shape=jax.ShapeDtypeStruct((N, 3 * C), jnp.float32),
)


def _gates_body(relu, gi_ref, gh_ref, x_ref, o_ref):
    gi = gi_ref[...]
    gh = gh_ref[...]
    h = x_ref[...]
    r = jax.nn.sigmoid(gi[:, 0:C] + gh[:, 0:C])
    z = jax.nn.sigmoid(gi[:, C:2 * C] + gh[:, C:2 * C])
    n = jnp.tanh(gi[:, 2 * C:] + r * gh[:, 2 * C:])
    out = (1.0 - z) * n + z * h
    if relu:
        out = jnp.maximum(out, 0.0)
    o_ref[...] = out


def _make_gates(relu):
    return pl.pallas_call(
        functools.partial(_gates_body, relu),
        grid=(5,),
        in_specs=[pl.BlockSpec((2000, 3 * C), lambda i: (i, 0)),
                  pl.BlockSpec((2000, 3 * C), lambda i: (i, 0)),
                  pl.BlockSpec((2000, C), lambda i: (i, 0))],
        out_specs=pl.BlockSpec((2000, C), lambda i: (i, 0)),
        out_shape=jax.ShapeDtypeStruct((N, C), jnp.float32),
    )


_gates = _make_gates(False)
_gates_relu = _make_gates(True)


def _build_plans(edge_index):
    src = edge_index[0]
    dst = edge_index[1]
    perm = jnp.argsort(dst, stable=True)
    ssrc = src[perm]
    sdst = dst[perm]

    bounds = jnp.asarray(_BOUNDS, jnp.int32)
    e = jnp.arange(E, dtype=jnp.int32)
    t_of_e = jnp.searchsorted(bounds, e, side="right").astype(jnp.int32) - 1
    pos = e - bounds[t_of_e]
    gpos = t_of_e * EPT_PAD + pos

    chunk_start = jnp.zeros((E,), jnp.bool_).at[bounds[:-1]].set(True)
    chunk_end = jnp.zeros((E,), jnp.bool_).at[bounds[1:] - 1].set(True)
    same = jnp.concatenate([jnp.zeros((1,), jnp.bool_),
                            sdst[1:] == sdst[:-1]]) & ~chunk_start
    islast = jnp.concatenate([sdst[1:] != sdst[:-1],
                              jnp.ones((1,), jnp.bool_)]) | chunk_end

    t = jnp.arange(NW, dtype=jnp.int32)
    bt = bounds[:-1]
    cont = (bt % (E // NSC) != 0) & (sdst[bt] == sdst[jnp.maximum(bt - 1, 0)])
    first_row = sdst[bt]
    sac = N + (t % NTILE)
    stash = N + NTILE + (t % NTILE)

    redirect = cont[t_of_e] & (sdst == first_row[t_of_e])
    target = jnp.where(redirect, stash[t_of_e], sdst)
    rowid_e = jnp.where(islast, target, sac[t_of_e])

    flat = NW * EPT_PAD
    fpos = jnp.arange(flat, dtype=jnp.int32)
    pad_t = fpos // EPT_PAD
    gidx = ((fpos * 2003) % N).at[gpos].set(ssrc)
    rowid = (N + (pad_t % NTILE)).at[gpos].set(rowid_e)
    same_p = jnp.zeros((flat,), jnp.int32).at[gpos].set(same.astype(jnp.int32))

    pk = jnp.stack([gidx.reshape(NW, NCH, K),
                    rowid.reshape(NW, NCH, K),
                    same_p.reshape(NW, NCH, K)], axis=2)
    frow = jnp.where(cont, first_row, sac).reshape(NSC, NTILE)
    return pk, frow


def kernel(x, edge_index,
           W1, wih1, whh1, bih1, bhh1,
           W2, wih2, whh2, bih2, bhh2,
           W3, wih3, whh3, bih3, bhh3):
    pk, frow = _build_plans(edge_index)
    zeros = jnp.zeros((NR, C), jnp.float32)

    layers = (
        (W1, wih1, whh1, bih1, bhh1, True),
        (W2, wih2, whh2, bih2, bhh2, True),
        (W3, wih3, whh3, bih3, bhh3, False),
    )
    for (W, wih, whh, bih, bhh, relu_at_end) in layers:
        bih2d = bih.reshape(1, 3 * C)
        bhh2d = bhh.reshape(1, 3 * C)
        for i in range(L):
            m = _mm(x, W[i])
            y = _sc_agg(m, pk, zeros, frow)
            gi = _gi(y[0], y[1], wih, bih2d)
            gh = _gh(x, whh, bhh2d)
            gates = _gates_relu if (relu_at_end and i == L - 1) else _gates
            x = gates(gi, gh, x)
    return x
